# baseline (speedup 1.0000x reference)
import functools

import jax
import jax.numpy as jnp
from jax import lax
from jax.experimental import pallas as pl
from jax.experimental.pallas import tpu as pltpu
from jax.experimental.pallas import tpu_sc as plsc

_B, _T = 4, 128
_Q = _B * _T
_DM = 256
_DK = 64
_DV = 128
_DE = 4
_DEP = 16
_N = 100000
_K = 32
_CH = 128
_NPAD = 102400
_NCH = _NPAD // _CH
_BN = 2048
_GRID = _NPAD // _BN
_CPB = _BN // _CH
_NEG = -2.0


def _scores_body(q_ref, kt_ref, s_ref, m_ref):
    i = pl.program_id(0)
    q = q_ref[...]
    kt = kt_ref[...]
    qk = jnp.dot(q, kt, preferred_element_type=jnp.float32)
    q2 = jnp.sum(q * q, axis=1, keepdims=True)
    k2 = jnp.sum(kt * kt, axis=0, keepdims=True)
    dist2 = (q2 + k2) - 2.0 * qk
    kern = jnp.exp(-dist2 / 128.0)

    def emit(kv):
        for c in range(_CPB):
            blk = kv[:, c * _CH:(c + 1) * _CH]
            s_ref[c] = blk
            m_ref[0, :, c:c + 1] = jnp.max(blk, axis=1, keepdims=True)

    emit(kern)

    @pl.when(i >= _N // _BN)
    def _mask_pad():
        col = lax.broadcasted_iota(jnp.int32, (_Q, _BN), 1) + i * _BN
        emit(jnp.where(col < _N, kern, -1.0))


def _scores_call(q, keys):
    return pl.pallas_call(
        _scores_body,
        grid=(_GRID,),
        in_specs=[
            pl.BlockSpec((_Q, _DK), lambda i: (0, 0)),
            pl.BlockSpec((_DK, _BN), lambda i: (0, i)),
        ],
        out_specs=[
            pl.BlockSpec((_CPB, _Q, _CH), lambda i: (i, 0, 0)),
            pl.BlockSpec((1, _Q, _CPB), lambda i: (i, 0, 0)),
        ],
        out_shape=[
            jax.ShapeDtypeStruct((_NCH, _Q, _CH), jnp.float32),
            jax.ShapeDtypeStruct((_GRID, _Q, _CPB), jnp.float32),
        ],
    )(q, keys)


def _chunk_topk_body(m_ref, fidx_ref, scr_ref):
    for i in range(_GRID):
        scr_ref[:, i * _CPB:(i + 1) * _CPB] = m_ref[i]
    qrow = lax.broadcasted_iota(jnp.int32, (_Q, 1), 0)
    colio = lax.broadcasted_iota(jnp.int32, (_Q, _NCH), 1)
    kio = lax.broadcasted_iota(jnp.int32, (_Q, _K), 1)
    big = jnp.int32(2 ** 30)

    def body(j, carry):
        s = scr_ref[...]
        mx = jnp.max(s, axis=1, keepdims=True)
        eq = s == mx
        cidx = jnp.min(jnp.where(eq, colio, big), axis=1, keepdims=True)
        fidx_ref[...] = jnp.where(kio == j, cidx * _Q + qrow, fidx_ref[...])
        scr_ref[...] = jnp.where(colio == cidx, _NEG, s)
        return carry

    lax.fori_loop(0, _K, body, 0)


def _chunk_topk_call(m):
    return pl.pallas_call(
        _chunk_topk_body,
        out_shape=jax.ShapeDtypeStruct((_Q, _K), jnp.int32),
        scratch_shapes=[pltpu.VMEM((_Q, _NCH), jnp.float32)],
    )(m)


def _final_topk_body(cand_ref, fidx_ref, vals_ref, gidx_ref, scr_ref, gscr_ref):
    qrow = lax.broadcasted_iota(jnp.int32, (_Q, 1), 0)
    lane = lax.broadcasted_iota(jnp.int32, (_Q, _CH), 1)
    for c in range(_K):
        scr_ref[:, c * _CH:(c + 1) * _CH] = cand_ref[:, c, :]
        chunk = fidx_ref[:, c:c + 1] // _Q
        gscr_ref[:, c * _CH:(c + 1) * _CH] = chunk * _CH + lane
    kio = lax.broadcasted_iota(jnp.int32, (_Q, _K), 1)
    big = jnp.int32(2 ** 30)

    def body(j, carry):
        s = scr_ref[...]
        gidx = gscr_ref[...]
        mx = jnp.max(s, axis=1, keepdims=True)
        eq = s == mx
        gsel = jnp.min(jnp.where(eq, gidx, big), axis=1, keepdims=True)
        vals_ref[...] = jnp.where(kio == j, mx, vals_ref[...])
        gidx_ref[...] = jnp.where(kio == j, gsel, gidx_ref[...])
        scr_ref[...] = jnp.where(gidx == gsel, _NEG, s)
        return carry

    lax.fori_loop(0, _K, body, 0)


def _final_topk_call(cand, fidx):
    return pl.pallas_call(
        _final_topk_body,
        out_shape=[
            jax.ShapeDtypeStruct((_Q, _K), jnp.float32),
            jax.ShapeDtypeStruct((_Q, _K), jnp.int32),
        ],
        scratch_shapes=[
            pltpu.VMEM((_Q, _K * _CH), jnp.float32),
            pltpu.VMEM((_Q, _K * _CH), jnp.int32),
        ],
    )(cand, fidx)


def _combine_body(vals_ref, v_ref, e_ref, x_ref, wgt_ref, bias_ref,
                  rv_ref, re_ref, g_ref):
    vals = vals_ref[...]
    w = vals / (jnp.sum(vals, axis=1, keepdims=True) + 1e-8)
    accv = jnp.sum(w[:, :, None] * v_ref[...], axis=1)
    acce = jnp.sum(w[:, :, None] * e_ref[...], axis=1)
    rv_ref[...] = accv
    re_ref[...] = acce[:, :_DE]
    wgt = wgt_ref[...]
    z1 = jnp.sum(x_ref[...] * wgt[:, :_DM], axis=1, keepdims=True)
    z2 = jnp.sum(accv * wgt[:, _DM:], axis=1, keepdims=True)
    z = z1 + z2 + bias_ref[...]
    g_ref[...] = 1.0 / (1.0 + jnp.exp(-z))


def _combine_call(vals, vsel, esel, x, wgt, bias):
    return pl.pallas_call(
        _combine_body,
        out_shape=[
            jax.ShapeDtypeStruct((_Q, _DV), jnp.float32),
            jax.ShapeDtypeStruct((_Q, _DE), jnp.float32),
            jax.ShapeDtypeStruct((_Q, 1), jnp.float32),
        ],
    )(vals, vsel, esel, x, wgt, bias)


_NC, _NS = 2, 16
_NW = _NC * _NS
_ROWS = _Q * _K
_RPW = _ROWS // _NW
_CHK = 128
_NCHK = _RPW // _CHK


def _gather_scores_call(s2, fidx):
    mesh = plsc.VectorSubcoreMesh(core_axis_name="c", subcore_axis_name="s")

    @functools.partial(
        pl.kernel,
        mesh=mesh,
        out_type=jax.ShapeDtypeStruct((_ROWS, _CH), jnp.float32),
        scratch_types=[
            pltpu.VMEM((_NCHK, _CHK), jnp.int32),
            pltpu.VMEM((_CHK, _CH), jnp.float32),
            pltpu.SemaphoreType.DMA,
        ],
    )
    def k(s2_hbm, idx_hbm, out_hbm, idx_v, rows_v, sem):
        wid = lax.axis_index("s") * _NC + lax.axis_index("c")
        base = wid * _RPW
        for t in range(_NCHK):
            pltpu.sync_copy(idx_hbm.at[pl.ds(base + t * _CHK, _CHK)],
                            idx_v.at[t])
            pltpu.async_copy(s2_hbm.at[idx_v.at[t]], rows_v, sem).wait()
            pltpu.sync_copy(rows_v, out_hbm.at[pl.ds(base + t * _CHK, _CHK)])

    return k(s2, fidx)


def _gather_ve_call(vtab, etab, gidx):
    mesh = plsc.VectorSubcoreMesh(core_axis_name="c", subcore_axis_name="s")

    @functools.partial(
        pl.kernel,
        mesh=mesh,
        compiler_params=pltpu.CompilerParams(use_tc_tiling_on_sc=False),
        out_type=[
            jax.ShapeDtypeStruct((_ROWS, _DV), jnp.float32),
            jax.ShapeDtypeStruct((_ROWS, _DEP), jnp.float32),
        ],
        scratch_types=[
            pltpu.VMEM((_NCHK, _CHK), jnp.int32),
            pltpu.VMEM((_CHK, _DV), jnp.float32),
            pltpu.VMEM((_CHK, _DEP), jnp.float32),
            pltpu.SemaphoreType.DMA,
        ],
    )
    def k(v_hbm, e_hbm, idx_hbm, outv_hbm, oute_hbm, idx_v, vr, er, sem):
        wid = lax.axis_index("s") * _NC + lax.axis_index("c")
        base = wid * _RPW
        for t in range(_NCHK):
            pltpu.sync_copy(idx_hbm.at[pl.ds(base + t * _CHK, _CHK)],
                            idx_v.at[t])
            pltpu.async_copy(v_hbm.at[idx_v.at[t]], vr, sem).wait()
            pltpu.async_copy(e_hbm.at[idx_v.at[t]], er, sem).wait()
            pltpu.sync_copy(vr, outv_hbm.at[pl.ds(base + t * _CHK, _CHK)])
            pltpu.sync_copy(er, oute_hbm.at[pl.ds(base + t * _CHK, _CHK)])

    return k(vtab, etab, gidx)


def kernel(x, q_tilde, g_prior, mem_keys, mem_values, mem_emotions,
           Wg_w, Wg_b, gate_prior_weight):
    q = q_tilde.reshape(_Q, _DK)
    kt = jnp.pad(mem_keys, ((0, _NPAD - _N), (0, 0))).T

    s3, m3 = _scores_call(q, kt)
    fidx = _chunk_topk_call(m3)
    cand = _gather_scores_call(s3.reshape(_NCH * _Q, _CH),
                               fidx.reshape(_ROWS))
    vals, gidx = _final_topk_call(cand.reshape(_Q, _K, _CH), fidx)

    epad = jnp.pad(mem_emotions, ((0, 0), (0, _DEP - _DE)))
    vsel, esel = _gather_ve_call(mem_values, epad, gidx.reshape(_ROWS))

    bias = Wg_b.reshape(1, 1) + gate_prior_weight * g_prior.reshape(_Q, 1)
    rv, re, g = _combine_call(vals,
                              vsel.reshape(_Q, _K, _DV),
                              esel.reshape(_Q, _K, _DEP),
                              x.reshape(_Q, _DM),
                              Wg_w.reshape(1, _DM + _DV),
                              bias)
    return (rv.reshape(_B, _T, _DV), re.reshape(_B, _T, _DE),
            g.reshape(_B, _T, 1))


# speedup vs baseline: 1.1842x; 1.1842x over previous
import functools

import jax
import jax.numpy as jnp
from jax import lax
from jax.experimental import pallas as pl
from jax.experimental.pallas import tpu as pltpu
from jax.experimental.pallas import tpu_sc as plsc

_B, _T = 4, 128
_Q = _B * _T
_DM = 256
_DK = 64
_DV = 128
_DE = 4
_DEP = 128
_N = 100000
_K = 32
_CH = 128
_NPAD = 102400
_NCH = _NPAD // _CH
_BN = 2048
_GRID = _NPAD // _BN
_CPB = _BN // _CH
_NEG = -2.0


def _scores_body(q_ref, kt_ref, s_ref, m_ref):
    i = pl.program_id(0)
    q = q_ref[...]
    kt = kt_ref[...]
    qk = jnp.dot(q, kt, preferred_element_type=jnp.float32)
    q2 = jnp.sum(q * q, axis=1, keepdims=True)
    k2 = jnp.sum(kt * kt, axis=0, keepdims=True)
    dist2 = (q2 + k2) - 2.0 * qk
    kern = jnp.exp(-dist2 / 128.0)

    def emit(kv):
        for c in range(_CPB):
            blk = kv[:, c * _CH:(c + 1) * _CH]
            s_ref[c] = blk
            m_ref[0, :, c:c + 1] = jnp.max(blk, axis=1, keepdims=True)

    emit(kern)

    @pl.when(i >= _N // _BN)
    def _mask_pad():
        col = lax.broadcasted_iota(jnp.int32, (_Q, _BN), 1) + i * _BN
        emit(jnp.where(col < _N, kern, -1.0))


def _scores_call(q, keys):
    return pl.pallas_call(
        _scores_body,
        grid=(_GRID,),
        in_specs=[
            pl.BlockSpec((_Q, _DK), lambda i: (0, 0)),
            pl.BlockSpec((_DK, _BN), lambda i: (0, i)),
        ],
        out_specs=[
            pl.BlockSpec((_CPB, _Q, _CH), lambda i: (i, 0, 0)),
            pl.BlockSpec((1, _Q, _CPB), lambda i: (i, 0, 0)),
        ],
        out_shape=[
            jax.ShapeDtypeStruct((_NCH, _Q, _CH), jnp.float32),
            jax.ShapeDtypeStruct((_GRID, _Q, _CPB), jnp.float32),
        ],
    )(q, keys)


def _chunk_topk_body(m_ref, fidx_ref, scr_ref):
    for i in range(_GRID):
        scr_ref[:, i * _CPB:(i + 1) * _CPB] = m_ref[i]
    qrow = lax.broadcasted_iota(jnp.int32, (_Q, 1), 0)
    colio = lax.broadcasted_iota(jnp.int32, (_Q, _NCH), 1)
    kio = lax.broadcasted_iota(jnp.int32, (_Q, _K), 1)
    big = jnp.int32(2 ** 30)

    def body(j, carry):
        s = scr_ref[...]
        mx = jnp.max(s, axis=1, keepdims=True)
        eq = s == mx
        cidx = jnp.min(jnp.where(eq, colio, big), axis=1, keepdims=True)
        fidx_ref[...] = jnp.where(kio == j, cidx * _Q + qrow, fidx_ref[...])
        scr_ref[...] = jnp.where(colio == cidx, _NEG, s)
        return carry

    lax.fori_loop(0, _K, body, 0)


def _chunk_topk_call(m):
    return pl.pallas_call(
        _chunk_topk_body,
        out_shape=jax.ShapeDtypeStruct((_Q, _K), jnp.int32),
        scratch_shapes=[pltpu.VMEM((_Q, _NCH), jnp.float32)],
    )(m)


def _final_topk_body(cand_ref, fidx_ref, vals_ref, gidx_ref, scr_ref, gscr_ref):
    qrow = lax.broadcasted_iota(jnp.int32, (_Q, 1), 0)
    lane = lax.broadcasted_iota(jnp.int32, (_Q, _CH), 1)
    for c in range(_K):
        scr_ref[:, c * _CH:(c + 1) * _CH] = cand_ref[:, c, :]
        chunk = fidx_ref[:, c:c + 1] // _Q
        gscr_ref[:, c * _CH:(c + 1) * _CH] = chunk * _CH + lane
    kio = lax.broadcasted_iota(jnp.int32, (_Q, _K), 1)
    big = jnp.int32(2 ** 30)

    def make_body(wl):
        def body(j, carry):
            s = scr_ref[:, :wl]
            gidx = gscr_ref[:, :wl]
            mx = jnp.max(s, axis=1, keepdims=True)
            eq = s == mx
            gsel = jnp.min(jnp.where(eq, gidx, big), axis=1, keepdims=True)
            vals_ref[...] = jnp.where(kio == j, mx, vals_ref[...])
            gidx_ref[...] = jnp.where(kio == j, gsel, gidx_ref[...])
            scr_ref[:, :wl] = jnp.where(gidx == gsel, _NEG, s)
            return carry
        return body

    for t in range(4):
        lax.fori_loop(t * 8, (t + 1) * 8, make_body((t + 1) * 8 * _CH), 0)


def _final_topk_call(cand, fidx):
    return pl.pallas_call(
        _final_topk_body,
        out_shape=[
            jax.ShapeDtypeStruct((_Q, _K), jnp.float32),
            jax.ShapeDtypeStruct((_Q, _K), jnp.int32),
        ],
        scratch_shapes=[
            pltpu.VMEM((_Q, _K * _CH), jnp.float32),
            pltpu.VMEM((_Q, _K * _CH), jnp.int32),
        ],
    )(cand, fidx)


def _combine_body(vals_ref, v_ref, e_ref, x_ref, wgt_ref, bias_ref,
                  rv_ref, re_ref, g_ref):
    vals = vals_ref[...]
    w = vals / (jnp.sum(vals, axis=1, keepdims=True) + 1e-8)
    accv = jnp.sum(w[:, :, None] * v_ref[...], axis=1)
    acce = jnp.sum(w[:, :, None] * e_ref[...], axis=1)
    rv_ref[...] = accv
    re_ref[...] = acce[:, :_DE]
    wgt = wgt_ref[...]
    z1 = jnp.sum(x_ref[...] * wgt[:, :_DM], axis=1, keepdims=True)
    z2 = jnp.sum(accv * wgt[:, _DM:], axis=1, keepdims=True)
    z = z1 + z2 + bias_ref[...]
    g_ref[...] = 1.0 / (1.0 + jnp.exp(-z))


def _combine_call(vals, vsel, esel, x, wgt, bias):
    return pl.pallas_call(
        _combine_body,
        out_shape=[
            jax.ShapeDtypeStruct((_Q, _DV), jnp.float32),
            jax.ShapeDtypeStruct((_Q, _DE), jnp.float32),
            jax.ShapeDtypeStruct((_Q, 1), jnp.float32),
        ],
    )(vals, vsel, esel, x, wgt, bias)


_NC, _NS = 2, 16
_NW = _NC * _NS
_ROWS = _Q * _K
_RPW = _ROWS // _NW
_CHK = 128
_NCHK = _RPW // _CHK


def _gather_scores_call(s2, fidx):
    mesh = plsc.VectorSubcoreMesh(core_axis_name="c", subcore_axis_name="s")

    @functools.partial(
        pl.kernel,
        mesh=mesh,
        out_type=jax.ShapeDtypeStruct((_ROWS, _CH), jnp.float32),
        scratch_types=[
            pltpu.VMEM((_NCHK, _CHK), jnp.int32),
            pltpu.VMEM((_CHK, _CH), jnp.float32),
            pltpu.SemaphoreType.DMA,
        ],
    )
    def k(s2_hbm, idx_hbm, out_hbm, idx_v, rows_v, sem):
        wid = lax.axis_index("s") * _NC + lax.axis_index("c")
        base = wid * _RPW
        for t in range(_NCHK):
            pltpu.sync_copy(idx_hbm.at[pl.ds(base + t * _CHK, _CHK)],
                            idx_v.at[t])
            pltpu.async_copy(s2_hbm.at[idx_v.at[t]], rows_v, sem).wait()
            pltpu.sync_copy(rows_v, out_hbm.at[pl.ds(base + t * _CHK, _CHK)])

    return k(s2, fidx)


def _gather_ve_call(vtab, etab, gidx):
    mesh = plsc.VectorSubcoreMesh(core_axis_name="c", subcore_axis_name="s")

    @functools.partial(
        pl.kernel,
        mesh=mesh,
        out_type=[
            jax.ShapeDtypeStruct((_ROWS, _DV), jnp.float32),
            jax.ShapeDtypeStruct((_ROWS, _DEP), jnp.float32),
        ],
        scratch_types=[
            pltpu.VMEM((_NCHK, _CHK), jnp.int32),
            pltpu.VMEM((_CHK, _DV), jnp.float32),
            pltpu.VMEM((_CHK, _DEP), jnp.float32),
            pltpu.SemaphoreType.DMA,
        ],
    )
    def k(v_hbm, e_hbm, idx_hbm, outv_hbm, oute_hbm, idx_v, vr, er, sem):
        wid = lax.axis_index("s") * _NC + lax.axis_index("c")
        base = wid * _RPW
        for t in range(_NCHK):
            pltpu.sync_copy(idx_hbm.at[pl.ds(base + t * _CHK, _CHK)],
                            idx_v.at[t])
            pltpu.async_copy(v_hbm.at[idx_v.at[t]], vr, sem).wait()
            pltpu.async_copy(e_hbm.at[idx_v.at[t]], er, sem).wait()
            pltpu.sync_copy(vr, outv_hbm.at[pl.ds(base + t * _CHK, _CHK)])
            pltpu.sync_copy(er, oute_hbm.at[pl.ds(base + t * _CHK, _CHK)])

    return k(vtab, etab, gidx)


def kernel(x, q_tilde, g_prior, mem_keys, mem_values, mem_emotions,
           Wg_w, Wg_b, gate_prior_weight):
    q = q_tilde.reshape(_Q, _DK)
    kt = jnp.pad(mem_keys, ((0, _NPAD - _N), (0, 0))).T

    s3, m3 = _scores_call(q, kt)
    fidx = _chunk_topk_call(m3)
    cand = _gather_scores_call(s3.reshape(_NCH * _Q, _CH),
                               fidx.reshape(_ROWS))
    vals, gidx = _final_topk_call(cand.reshape(_Q, _K, _CH), fidx)

    epad = jnp.pad(mem_emotions, ((0, 0), (0, _DEP - _DE)))
    vsel, esel = _gather_ve_call(mem_values, epad, gidx.reshape(_ROWS))

    bias = Wg_b.reshape(1, 1) + gate_prior_weight * g_prior.reshape(_Q, 1)
    rv, re, g = _combine_call(vals,
                              vsel.reshape(_Q, _K, _DV),
                              esel.reshape(_Q, _K, _DEP),
                              x.reshape(_Q, _DM),
                              Wg_w.reshape(1, _DM + _DV),
                              bias)
    return (rv.reshape(_B, _T, _DV), re.reshape(_B, _T, _DE),
            g.reshape(_B, _T, 1))


# speedup vs baseline: 1.2575x; 1.0620x over previous
import functools

import jax
import jax.numpy as jnp
from jax import lax
from jax.experimental import pallas as pl
from jax.experimental.pallas import tpu as pltpu
from jax.experimental.pallas import tpu_sc as plsc

_B, _T = 4, 128
_Q = _B * _T
_DM = 256
_DK = 64
_DV = 128
_DE = 4
_DEP = 128
_N = 100000
_K = 32
_CH = 128
_NPAD = 100352
_NCH = _NPAD // _CH
_BN = 2048
_GRID = _NPAD // _BN
_CPB = _BN // _CH
_NEG = -2.0


def _scores_body(q_ref, kt_ref, s_ref, m_ref):
    i = pl.program_id(0)
    q = q_ref[...]
    kt = kt_ref[...]
    qk = jnp.dot(q, kt, preferred_element_type=jnp.float32)
    q2 = jnp.sum(q * q, axis=1, keepdims=True)
    k2 = jnp.sum(kt * kt, axis=0, keepdims=True)
    dist2 = (q2 + k2) - 2.0 * qk
    kern = jnp.exp(-dist2 / 128.0)

    def emit(kv):
        for c in range(_CPB):
            blk = kv[:, c * _CH:(c + 1) * _CH]
            s_ref[c] = blk
            m_ref[0, :, c:c + 1] = jnp.max(blk, axis=1, keepdims=True)

    emit(kern)

    @pl.when(i >= _N // _BN)
    def _mask_pad():
        col = lax.broadcasted_iota(jnp.int32, (_Q, _BN), 1) + i * _BN
        emit(jnp.where(col < _N, kern, -1.0))


def _scores_call(q, keys):
    return pl.pallas_call(
        _scores_body,
        grid=(_GRID,),
        in_specs=[
            pl.BlockSpec((_Q, _DK), lambda i: (0, 0)),
            pl.BlockSpec((_DK, _BN), lambda i: (0, i)),
        ],
        out_specs=[
            pl.BlockSpec((_CPB, _Q, _CH), lambda i: (i, 0, 0)),
            pl.BlockSpec((1, _Q, _CPB), lambda i: (i, 0, 0)),
        ],
        out_shape=[
            jax.ShapeDtypeStruct((_NCH, _Q, _CH), jnp.float32),
            jax.ShapeDtypeStruct((_GRID, _Q, _CPB), jnp.float32),
        ],
    )(q, keys)


def _chunk_topk_body(m_ref, fidx_ref, scr_ref):
    for i in range(_GRID):
        scr_ref[:, i * _CPB:(i + 1) * _CPB] = m_ref[i]
    qrow = lax.broadcasted_iota(jnp.int32, (_Q, 1), 0)
    colio = lax.broadcasted_iota(jnp.int32, (_Q, _NCH), 1)
    kio = lax.broadcasted_iota(jnp.int32, (_Q, _K), 1)
    big = jnp.int32(2 ** 30)

    def body(j, carry):
        s = scr_ref[...]
        mx = jnp.max(s, axis=1, keepdims=True)
        eq = s == mx
        cidx = jnp.min(jnp.where(eq, colio, big), axis=1, keepdims=True)
        fidx_ref[...] = jnp.where(kio == j, cidx * _Q + qrow, fidx_ref[...])
        scr_ref[...] = jnp.where(colio == cidx, _NEG, s)
        return carry

    lax.fori_loop(0, _K, body, 0)


def _chunk_topk_call(m):
    return pl.pallas_call(
        _chunk_topk_body,
        out_shape=jax.ShapeDtypeStruct((_Q, _K), jnp.int32),
        scratch_shapes=[pltpu.VMEM((_Q, _NCH), jnp.float32)],
    )(m)


def _final_topk_body(cand_ref, fidx_ref, vals_ref, gidx_ref, scr_ref, gscr_ref):
    qrow = lax.broadcasted_iota(jnp.int32, (_Q, 1), 0)
    lane = lax.broadcasted_iota(jnp.int32, (_Q, _CH), 1)
    for c in range(_K):
        scr_ref[:, c * _CH:(c + 1) * _CH] = cand_ref[:, c, :]
        chunk = fidx_ref[:, c:c + 1] // _Q
        gscr_ref[:, c * _CH:(c + 1) * _CH] = chunk * _CH + lane
    kio = lax.broadcasted_iota(jnp.int32, (_Q, _K), 1)
    big = jnp.int32(2 ** 30)

    def make_body(wl):
        def body(j, carry):
            s = scr_ref[:, :wl]
            gidx = gscr_ref[:, :wl]
            mx = jnp.max(s, axis=1, keepdims=True)
            eq = s == mx
            gsel = jnp.min(jnp.where(eq, gidx, big), axis=1, keepdims=True)
            vals_ref[...] = jnp.where(kio == j, mx, vals_ref[...])
            gidx_ref[...] = jnp.where(kio == j, gsel, gidx_ref[...])
            scr_ref[:, :wl] = jnp.where(gidx == gsel, _NEG, s)
            return carry
        return body

    for t in range(4):
        lax.fori_loop(t * 8, (t + 1) * 8, make_body((t + 1) * 8 * _CH), 0)


def _final_topk_call(cand, fidx):
    return pl.pallas_call(
        _final_topk_body,
        out_shape=[
            jax.ShapeDtypeStruct((_Q, _K), jnp.float32),
            jax.ShapeDtypeStruct((_Q, _K), jnp.int32),
        ],
        scratch_shapes=[
            pltpu.VMEM((_Q, _K * _CH), jnp.float32),
            pltpu.VMEM((_Q, _K * _CH), jnp.int32),
        ],
    )(cand, fidx)


def _combine_body(vals_ref, v_ref, e_ref, x_ref, wgt_ref, bias_ref,
                  rv_ref, re_ref, g_ref):
    vals = vals_ref[...]
    w = vals / (jnp.sum(vals, axis=1, keepdims=True) + 1e-8)
    accv = jnp.sum(w[:, :, None] * v_ref[...], axis=1)
    acce = jnp.sum(w[:, :, None] * e_ref[...], axis=1)
    rv_ref[...] = accv
    re_ref[...] = acce[:, :_DE]
    wgt = wgt_ref[...]
    z1 = jnp.sum(x_ref[...] * wgt[:, :_DM], axis=1, keepdims=True)
    z2 = jnp.sum(accv * wgt[:, _DM:], axis=1, keepdims=True)
    z = z1 + z2 + bias_ref[...]
    g_ref[...] = 1.0 / (1.0 + jnp.exp(-z))


def _combine_call(vals, vsel, esel, x, wgt, bias):
    return pl.pallas_call(
        _combine_body,
        out_shape=[
            jax.ShapeDtypeStruct((_Q, _DV), jnp.float32),
            jax.ShapeDtypeStruct((_Q, _DE), jnp.float32),
            jax.ShapeDtypeStruct((_Q, 1), jnp.float32),
        ],
    )(vals, vsel, esel, x, wgt, bias)


_NC, _NS = 2, 16
_NW = _NC * _NS
_ROWS = _Q * _K
_RPW = _ROWS // _NW
_CHK = 128
_NCHK = _RPW // _CHK


def _gather_scores_call(s2, fidx):
    mesh = plsc.VectorSubcoreMesh(core_axis_name="c", subcore_axis_name="s")

    @functools.partial(
        pl.kernel,
        mesh=mesh,
        out_type=jax.ShapeDtypeStruct((_ROWS, _CH), jnp.float32),
        scratch_types=[
            pltpu.VMEM((_NCHK, _CHK), jnp.int32),
            pltpu.VMEM((_CHK, _CH), jnp.float32),
            pltpu.SemaphoreType.DMA,
        ],
    )
    def k(s2_hbm, idx_hbm, out_hbm, idx_v, rows_v, sem):
        wid = lax.axis_index("s") * _NC + lax.axis_index("c")
        base = wid * _RPW
        for t in range(_NCHK):
            pltpu.sync_copy(idx_hbm.at[pl.ds(base + t * _CHK, _CHK)],
                            idx_v.at[t])
            pltpu.async_copy(s2_hbm.at[idx_v.at[t]], rows_v, sem).wait()
            pltpu.sync_copy(rows_v, out_hbm.at[pl.ds(base + t * _CHK, _CHK)])

    return k(s2, fidx)


def _gather_ve_call(vtab, etab, gidx):
    mesh = plsc.VectorSubcoreMesh(core_axis_name="c", subcore_axis_name="s")

    @functools.partial(
        pl.kernel,
        mesh=mesh,
        out_type=[
            jax.ShapeDtypeStruct((_ROWS, _DV), jnp.float32),
            jax.ShapeDtypeStruct((_ROWS, _DEP), jnp.float32),
        ],
        scratch_types=[
            pltpu.VMEM((_NCHK, _CHK), jnp.int32),
            pltpu.VMEM((_CHK, _DV), jnp.float32),
            pltpu.VMEM((_CHK, _DEP), jnp.float32),
            pltpu.SemaphoreType.DMA,
        ],
    )
    def k(v_hbm, e_hbm, idx_hbm, outv_hbm, oute_hbm, idx_v, vr, er, sem):
        wid = lax.axis_index("s") * _NC + lax.axis_index("c")
        base = wid * _RPW
        for t in range(_NCHK):
            pltpu.sync_copy(idx_hbm.at[pl.ds(base + t * _CHK, _CHK)],
                            idx_v.at[t])
            pltpu.async_copy(v_hbm.at[idx_v.at[t]], vr, sem).wait()
            pltpu.async_copy(e_hbm.at[idx_v.at[t]], er, sem).wait()
            pltpu.sync_copy(vr, outv_hbm.at[pl.ds(base + t * _CHK, _CHK)])
            pltpu.sync_copy(er, oute_hbm.at[pl.ds(base + t * _CHK, _CHK)])

    return k(vtab, etab, gidx)


def kernel(x, q_tilde, g_prior, mem_keys, mem_values, mem_emotions,
           Wg_w, Wg_b, gate_prior_weight):
    q = q_tilde.reshape(_Q, _DK)
    kt = mem_keys.T

    s3, m3 = _scores_call(q, kt)
    fidx = _chunk_topk_call(m3)
    cand = _gather_scores_call(s3.reshape(_NCH * _Q, _CH),
                               fidx.reshape(_ROWS))
    vals, gidx = _final_topk_call(cand.reshape(_Q, _K, _CH), fidx)

    epad = jnp.pad(mem_emotions, ((0, 0), (0, _DEP - _DE)))
    vsel, esel = _gather_ve_call(mem_values, epad, gidx.reshape(_ROWS))

    bias = Wg_b.reshape(1, 1) + gate_prior_weight * g_prior.reshape(_Q, 1)
    rv, re, g = _combine_call(vals,
                              vsel.reshape(_Q, _K, _DV),
                              esel.reshape(_Q, _K, _DEP),
                              x.reshape(_Q, _DM),
                              Wg_w.reshape(1, _DM + _DV),
                              bias)
    return (rv.reshape(_B, _T, _DV), re.reshape(_B, _T, _DE),
            g.reshape(_B, _T, 1))
